# baseline (speedup 1.0000x reference)
import functools

import jax
import jax.numpy as jnp
from jax import lax
from jax.experimental import pallas as pl
from jax.experimental.pallas import tpu as pltpu
from jax.experimental.pallas import tpu_sc as plsc

N = 10000
E = 320000
F = 128
NC = 2
NS = 16
LANES = 16
NW = NC * NS

CHUNK = 80
RPW = 128
T8 = RPW // 8
EPAD = NW * RPW * CHUNK

NPAD = 10240
ROWS_PT = NPAD // NS
ZB = 80


def _matmul_body(x_ref, wt_ref, o_ref):
    o_ref[...] = jnp.dot(x_ref[...], wt_ref[...],
                         preferred_element_type=jnp.float32)


def _combine_body(p_ref, b_ref, o_ref):
    o_ref[...] = p_ref[0] + p_ref[1] + b_ref[...][None, :]


def _scatter_body(xlin, ed_h, out_h, *scr):
    acc = scr[0]
    eds = scr[1:9]
    rows = scr[9:13]
    sis = scr[13:21]
    sgs = scr[21:25]
    sss = scr[25:29]

    c = lax.axis_index("c")
    s = lax.axis_index("s")
    w = c * NS + s
    rbase = w * RPW

    def _zero_rows(r, _):
        for j in range(F // LANES):
            rows[0][r, pl.ds(j * LANES, LANES)] = jnp.zeros(
                (LANES,), jnp.float32)
        return 0
    lax.fori_loop(0, ZB, _zero_rows, 0)
    for t in range(ROWS_PT // ZB):
        pltpu.sync_copy(rows[0].at[pl.ds(0, ZB)],
                        acc.at[pl.ds(s * ROWS_PT + t * ZB, ZB)])
    plsc.subcore_barrier()

    def _idx_start(j, k):
        pltpu.async_copy(ed_h.at[rbase + k], eds[j], sis[j])

    def _idx_wait(j, k):
        pltpu.make_async_copy(ed_h.at[rbase + k], eds[j], sis[j]).wait()

    def _scale(r, j):
        rref = rows[r]
        wref = eds[j]

        @plsc.parallel_loop(0, CHUNK // LANES)
        def _grp(g2):
            eb2 = g2 * LANES
            wvec = jax.lax.bitcast_convert_type(
                wref[2, pl.ds(eb2, LANES)], jnp.float32)
            for l in range(LANES):
                ew = wvec[l]
                e = eb2 + l
                for fj in range(F // LANES):
                    sl = pl.ds(fj * LANES, LANES)
                    rref[e, sl] = rref[e, sl] * ew

    def _ss_wait(r, j):
        pltpu.make_async_copy(rows[r], acc.at[eds[j].at[1]], sss[r]).wait()

    for j in range(6):
        _idx_start(j, j)
    for j in range(2):
        _idx_wait(j, j)
        pltpu.async_copy(xlin.at[eds[j].at[0]], rows[j], sgs[j])

    def _iter(t8, _):
        for m in range(8):
            rs = m % 4

            if m < 2:
                @pl.when(t8 > 0)
                def _():
                    _ss_wait((m + 2) % 4, (m + 6) % 8)
            else:
                _ss_wait((m + 2) % 4, (m + 6) % 8)

            if m < 2:
                _idx_start((m + 6) % 8, 8 * t8 + m + 6)
            else:
                @pl.when(t8 < T8 - 1)
                def _():
                    _idx_start((m + 6) % 8, 8 * t8 + m + 6)

            if m < 6:
                _idx_wait((m + 2) % 8, 8 * t8 + m + 2)
                pltpu.async_copy(xlin.at[eds[(m + 2) % 8].at[0]],
                                 rows[(m + 2) % 4], sgs[(m + 2) % 4])
            else:
                @pl.when(t8 < T8 - 1)
                def _():
                    _idx_wait((m + 2) % 8, 8 * t8 + m + 2)
                    pltpu.async_copy(xlin.at[eds[(m + 2) % 8].at[0]],
                                     rows[(m + 2) % 4], sgs[(m + 2) % 4])

            pltpu.make_async_copy(xlin.at[eds[m].at[0]], rows[rs],
                                  sgs[rs]).wait()
            _scale(rs, m)
            pltpu.async_copy(rows[rs], acc.at[eds[m].at[1]], sss[rs],
                             add=True)
        return 0
    lax.fori_loop(0, T8, _iter, 0)
    _ss_wait(2, 6)
    _ss_wait(3, 7)
    plsc.subcore_barrier()

    pltpu.sync_copy(acc.at[pl.ds(s * ROWS_PT, ROWS_PT)],
                    out_h.at[c, pl.ds(s * ROWS_PT, ROWS_PT)])


_scatter_kernel = functools.partial(
    pl.kernel,
    out_type=jax.ShapeDtypeStruct((NC, NPAD, F), jnp.float32),
    mesh=plsc.VectorSubcoreMesh(core_axis_name="c", subcore_axis_name="s"),
    scratch_types=(
        [pltpu.VMEM_SHARED((NPAD, F), jnp.float32)]
        + [pltpu.VMEM((3, CHUNK), jnp.int32) for _ in range(8)]
        + [pltpu.VMEM((CHUNK, F), jnp.float32) for _ in range(4)]
        + [pltpu.SemaphoreType.DMA for _ in range(16)]
    ),
)(_scatter_body)


@jax.jit
def kernel(x, edge_index, edge_weight, node_lock, W, b):
    x_lin = pl.pallas_call(
        _matmul_body,
        grid=(10,),
        in_specs=[
            pl.BlockSpec((N // 10, F), lambda i: (i, 0)),
            pl.BlockSpec((F, F), lambda i: (0, 0)),
        ],
        out_specs=pl.BlockSpec((N // 10, F), lambda i: (i, 0)),
        out_shape=jax.ShapeDtypeStruct((N, F), jnp.float32),
    )(x, W.T)

    npad_e = EPAD - E
    pad_idx = jnp.arange(npad_e, dtype=jnp.int32) % N
    srcp = jnp.concatenate([edge_index[0], pad_idx])
    dstp = jnp.concatenate([edge_index[1], pad_idx])
    wp = jnp.concatenate([edge_weight, jnp.zeros((npad_e,), jnp.float32)])
    wbits = jax.lax.bitcast_convert_type(wp, jnp.int32)
    nrows = EPAD // CHUNK
    edata = jnp.concatenate(
        [srcp.reshape(nrows, 1, CHUNK), dstp.reshape(nrows, 1, CHUNK),
         wbits.reshape(nrows, 1, CHUNK)], axis=1)

    partials = _scatter_kernel(x_lin, edata)

    out = pl.pallas_call(
        _combine_body,
        grid=(10,),
        in_specs=[
            pl.BlockSpec((NC, N // 10, F), lambda i: (0, i, 0)),
            pl.BlockSpec((F,), lambda i: (0,)),
        ],
        out_specs=pl.BlockSpec((N // 10, F), lambda i: (i, 0)),
        out_shape=jax.ShapeDtypeStruct((N, F), jnp.float32),
    )(partials, b)

    return (out, (edge_index, edge_weight))


# speedup vs baseline: 1.1222x; 1.1222x over previous
import functools

import jax
import jax.numpy as jnp
from jax import lax
from jax.experimental import pallas as pl
from jax.experimental.pallas import tpu as pltpu
from jax.experimental.pallas import tpu_sc as plsc

N = 10000
E = 320000
F = 128
NC = 2
NS = 16
LANES = 16
NW = NC * NS

CHUNK = 112
RPW = 90
T6 = RPW // 6
EPAD = NW * RPW * CHUNK

NPAD = 10240
ROWS_PT = NPAD // NS
ZB = 80


def _matmul_body(x_ref, wt_ref, o_ref):
    o_ref[...] = jnp.dot(x_ref[...], wt_ref[...],
                         preferred_element_type=jnp.float32)


def _combine_body(p_ref, b_ref, o_ref):
    o_ref[...] = p_ref[0] + p_ref[1] + b_ref[...][None, :]


def _scatter_body(xlin, src_h, dst_h, w_h, out_h, *scr):
    acc = scr[0]
    srcs = scr[1:7]
    dsts = scr[7:13]
    ws = scr[13:19]
    rows = scr[19:22]
    sis = scr[22:28]
    sgs = scr[28:31]
    sss = scr[31:34]

    c = lax.axis_index("c")
    s = lax.axis_index("s")
    w = c * NS + s
    ebase = w * RPW * CHUNK

    def _zero_rows(r, _):
        for j in range(F // LANES):
            rows[0][r, pl.ds(j * LANES, LANES)] = jnp.zeros(
                (LANES,), jnp.float32)
        return 0
    lax.fori_loop(0, ZB, _zero_rows, 0)
    for t in range(ROWS_PT // ZB):
        pltpu.sync_copy(rows[0].at[pl.ds(0, ZB)],
                        acc.at[pl.ds(s * ROWS_PT + t * ZB, ZB)])
    plsc.subcore_barrier()

    def _idx_start(j, k):
        eo = ebase + k * CHUNK
        pltpu.async_copy(src_h.at[pl.ds(eo, CHUNK)], srcs[j], sis[j])
        pltpu.async_copy(dst_h.at[pl.ds(eo, CHUNK)], dsts[j], sis[j])
        pltpu.async_copy(w_h.at[pl.ds(eo, CHUNK)], ws[j], sis[j])

    def _idx_wait(j, k):
        eo = ebase + k * CHUNK
        pltpu.make_async_copy(src_h.at[pl.ds(eo, CHUNK)], srcs[j],
                              sis[j]).wait()
        pltpu.make_async_copy(dst_h.at[pl.ds(eo, CHUNK)], dsts[j],
                              sis[j]).wait()
        pltpu.make_async_copy(w_h.at[pl.ds(eo, CHUNK)], ws[j],
                              sis[j]).wait()

    def _scale(r, j):
        rref = rows[r]
        wref = ws[j]

        @plsc.parallel_loop(0, CHUNK // LANES)
        def _grp(g2):
            eb2 = g2 * LANES
            wvec = wref[pl.ds(eb2, LANES)]
            for l in range(LANES):
                ew = wvec[l]
                e = eb2 + l
                for fj in range(F // LANES):
                    sl = pl.ds(fj * LANES, LANES)
                    rref[e, sl] = rref[e, sl] * ew

    def _ss_wait(r, j):
        pltpu.make_async_copy(rows[r], acc.at[dsts[j]], sss[r]).wait()

    for j in range(4):
        _idx_start(j, j)
    _idx_wait(0, 0)
    pltpu.async_copy(xlin.at[srcs[0]], rows[0], sgs[0])

    def _iter(t6, _):
        for m in range(6):
            rs = m % 3

            if m < 2:
                @pl.when(t6 > 0)
                def _():
                    _ss_wait((m + 1) % 3, (m + 4) % 6)
            else:
                _ss_wait((m + 1) % 3, (m + 4) % 6)

            if m < 2:
                _idx_start((m + 4) % 6, 6 * t6 + m + 4)
            else:
                @pl.when(t6 < T6 - 1)
                def _():
                    _idx_start((m + 4) % 6, 6 * t6 + m + 4)

            if m < 5:
                _idx_wait((m + 1) % 6, 6 * t6 + m + 1)
                pltpu.async_copy(xlin.at[srcs[(m + 1) % 6]],
                                 rows[(m + 1) % 3], sgs[(m + 1) % 3])
            else:
                @pl.when(t6 < T6 - 1)
                def _():
                    _idx_wait((m + 1) % 6, 6 * t6 + m + 1)
                    pltpu.async_copy(xlin.at[srcs[(m + 1) % 6]],
                                     rows[(m + 1) % 3], sgs[(m + 1) % 3])

            pltpu.make_async_copy(xlin.at[srcs[m]], rows[rs],
                                  sgs[rs]).wait()
            _scale(rs, m)
            pltpu.async_copy(rows[rs], acc.at[dsts[m]], sss[rs], add=True)
        return 0
    lax.fori_loop(0, T6, _iter, 0)
    _ss_wait(1, 4)
    _ss_wait(2, 5)
    plsc.subcore_barrier()

    pltpu.sync_copy(acc.at[pl.ds(s * ROWS_PT, ROWS_PT)],
                    out_h.at[c, pl.ds(s * ROWS_PT, ROWS_PT)])


_scatter_kernel = functools.partial(
    pl.kernel,
    out_type=jax.ShapeDtypeStruct((NC, NPAD, F), jnp.float32),
    mesh=plsc.VectorSubcoreMesh(core_axis_name="c", subcore_axis_name="s"),
    scratch_types=(
        [pltpu.VMEM_SHARED((NPAD, F), jnp.float32)]
        + [pltpu.VMEM((CHUNK,), jnp.int32) for _ in range(12)]
        + [pltpu.VMEM((CHUNK,), jnp.float32) for _ in range(6)]
        + [pltpu.VMEM((CHUNK, F), jnp.float32) for _ in range(3)]
        + [pltpu.SemaphoreType.DMA for _ in range(12)]
    ),
)(_scatter_body)


@jax.jit
def kernel(x, edge_index, edge_weight, node_lock, W, b):
    x_lin = pl.pallas_call(
        _matmul_body,
        grid=(10,),
        in_specs=[
            pl.BlockSpec((N // 10, F), lambda i: (i, 0)),
            pl.BlockSpec((F, F), lambda i: (0, 0)),
        ],
        out_specs=pl.BlockSpec((N // 10, F), lambda i: (i, 0)),
        out_shape=jax.ShapeDtypeStruct((N, F), jnp.float32),
    )(x, W.T)

    npad_e = EPAD - E
    pad_idx = jnp.arange(npad_e, dtype=jnp.int32) % N
    srcp = jnp.concatenate([edge_index[0], pad_idx])
    dstp = jnp.concatenate([edge_index[1], pad_idx])
    wp = jnp.concatenate([edge_weight, jnp.zeros((npad_e,), jnp.float32)])

    partials = _scatter_kernel(x_lin, srcp, dstp, wp)

    out = pl.pallas_call(
        _combine_body,
        grid=(10,),
        in_specs=[
            pl.BlockSpec((NC, N // 10, F), lambda i: (0, i, 0)),
            pl.BlockSpec((F,), lambda i: (0,)),
        ],
        out_specs=pl.BlockSpec((N // 10, F), lambda i: (i, 0)),
        out_shape=jax.ShapeDtypeStruct((N, F), jnp.float32),
    )(partials, b)

    return (out, (edge_index, edge_weight))


# speedup vs baseline: 1.1229x; 1.0006x over previous
import functools

import jax
import jax.numpy as jnp
from jax import lax
from jax.experimental import pallas as pl
from jax.experimental.pallas import tpu as pltpu
from jax.experimental.pallas import tpu_sc as plsc

N = 10000
E = 320000
F = 128
NC = 2
NS = 16
LANES = 16
NW = NC * NS

CHUNK = 112
RPW = 90
T6 = RPW // 6
EPAD = NW * RPW * CHUNK

NPAD = 10240
ROWS_PT = NPAD // NS
ZB = 80


def _matmul_body(x_ref, wt_ref, o_ref):
    o_ref[...] = jnp.dot(x_ref[...], wt_ref[...],
                         preferred_element_type=jnp.float32)


def _combine_body(p_ref, b_ref, o_ref):
    o_ref[...] = p_ref[0] + p_ref[1] + b_ref[...][None, :]


def _scatter_body(xlin, src_h, dst_h, w_h, out_h, *scr):
    acc = scr[0]
    srcs = scr[1:7]
    dsts = scr[7:13]
    ws = scr[13:19]
    rows = scr[19:22]
    sis = scr[22:28]
    sgs = scr[28:31]
    sss = scr[31:34]

    c = lax.axis_index("c")
    s = lax.axis_index("s")
    w = c * NS + s
    ebase = w * RPW * CHUNK

    def _idx_start(j, k):
        eo = ebase + k * CHUNK
        pltpu.async_copy(src_h.at[pl.ds(eo, CHUNK)], srcs[j], sis[j])
        pltpu.async_copy(dst_h.at[pl.ds(eo, CHUNK)], dsts[j], sis[j])
        pltpu.async_copy(w_h.at[pl.ds(eo, CHUNK)], ws[j], sis[j])

    def _idx_wait(j, k):
        eo = ebase + k * CHUNK
        pltpu.make_async_copy(src_h.at[pl.ds(eo, CHUNK)], srcs[j],
                              sis[j]).wait()
        pltpu.make_async_copy(dst_h.at[pl.ds(eo, CHUNK)], dsts[j],
                              sis[j]).wait()
        pltpu.make_async_copy(w_h.at[pl.ds(eo, CHUNK)], ws[j],
                              sis[j]).wait()

    def _scale(r, j):
        rref = rows[r]
        wref = ws[j]

        @plsc.parallel_loop(0, CHUNK // LANES)
        def _grp(g2):
            eb2 = g2 * LANES
            wvec = wref[pl.ds(eb2, LANES)]
            for l in range(LANES):
                ew = wvec[l]
                e = eb2 + l
                for fj in range(F // LANES):
                    sl = pl.ds(fj * LANES, LANES)
                    rref[e, sl] = rref[e, sl] * ew

    def _ss_wait(r, j):
        pltpu.make_async_copy(rows[r], acc.at[dsts[j]], sss[r]).wait()

    for j in range(4):
        _idx_start(j, j)

    def _zero_rows(r, _):
        for j in range(F // LANES):
            rows[1][r, pl.ds(j * LANES, LANES)] = jnp.zeros(
                (LANES,), jnp.float32)
        return 0
    lax.fori_loop(0, ZB, _zero_rows, 0)
    for t in range(ROWS_PT // ZB):
        pltpu.async_copy(rows[1].at[pl.ds(0, ZB)],
                         acc.at[pl.ds(s * ROWS_PT + t * ZB, ZB)], sss[0])
    _idx_wait(0, 0)
    for t in range(ROWS_PT // ZB):
        pltpu.make_async_copy(
            rows[1].at[pl.ds(0, ZB)],
            acc.at[pl.ds(s * ROWS_PT + t * ZB, ZB)], sss[0]).wait()
    plsc.subcore_barrier()
    pltpu.async_copy(xlin.at[srcs[0]], rows[0], sgs[0])

    def _iter(t6, _):
        for m in range(6):
            rs = m % 3

            if m < 2:
                @pl.when(t6 > 0)
                def _():
                    _ss_wait((m + 1) % 3, (m + 4) % 6)
            else:
                _ss_wait((m + 1) % 3, (m + 4) % 6)

            if m < 2:
                _idx_start((m + 4) % 6, 6 * t6 + m + 4)
            else:
                @pl.when(t6 < T6 - 1)
                def _():
                    _idx_start((m + 4) % 6, 6 * t6 + m + 4)

            if m < 5:
                _idx_wait((m + 1) % 6, 6 * t6 + m + 1)
                pltpu.async_copy(xlin.at[srcs[(m + 1) % 6]],
                                 rows[(m + 1) % 3], sgs[(m + 1) % 3])
            else:
                @pl.when(t6 < T6 - 1)
                def _():
                    _idx_wait((m + 1) % 6, 6 * t6 + m + 1)
                    pltpu.async_copy(xlin.at[srcs[(m + 1) % 6]],
                                     rows[(m + 1) % 3], sgs[(m + 1) % 3])

            pltpu.make_async_copy(xlin.at[srcs[m]], rows[rs],
                                  sgs[rs]).wait()
            _scale(rs, m)
            pltpu.async_copy(rows[rs], acc.at[dsts[m]], sss[rs], add=True)
        return 0
    lax.fori_loop(0, T6, _iter, 0)
    _ss_wait(1, 4)
    _ss_wait(2, 5)
    plsc.subcore_barrier()

    pltpu.sync_copy(acc.at[pl.ds(s * ROWS_PT, ROWS_PT)],
                    out_h.at[c, pl.ds(s * ROWS_PT, ROWS_PT)])


_scatter_kernel = functools.partial(
    pl.kernel,
    out_type=jax.ShapeDtypeStruct((NC, NPAD, F), jnp.float32),
    mesh=plsc.VectorSubcoreMesh(core_axis_name="c", subcore_axis_name="s"),
    scratch_types=(
        [pltpu.VMEM_SHARED((NPAD, F), jnp.float32)]
        + [pltpu.VMEM((CHUNK,), jnp.int32) for _ in range(12)]
        + [pltpu.VMEM((CHUNK,), jnp.float32) for _ in range(6)]
        + [pltpu.VMEM((CHUNK, F), jnp.float32) for _ in range(3)]
        + [pltpu.SemaphoreType.DMA for _ in range(12)]
    ),
)(_scatter_body)


@jax.jit
def kernel(x, edge_index, edge_weight, node_lock, W, b):
    x_lin = pl.pallas_call(
        _matmul_body,
        grid=(10,),
        in_specs=[
            pl.BlockSpec((N // 10, F), lambda i: (i, 0)),
            pl.BlockSpec((F, F), lambda i: (0, 0)),
        ],
        out_specs=pl.BlockSpec((N // 10, F), lambda i: (i, 0)),
        out_shape=jax.ShapeDtypeStruct((N, F), jnp.float32),
    )(x, W.T)

    npad_e = EPAD - E
    pad_idx = jnp.arange(npad_e, dtype=jnp.int32) % N
    srcp = jnp.concatenate([edge_index[0], pad_idx])
    dstp = jnp.concatenate([edge_index[1], pad_idx])
    wp = jnp.concatenate([edge_weight, jnp.zeros((npad_e,), jnp.float32)])

    partials = _scatter_kernel(x_lin, srcp, dstp, wp)

    out = pl.pallas_call(
        _combine_body,
        grid=(10,),
        in_specs=[
            pl.BlockSpec((NC, N // 10, F), lambda i: (0, i, 0)),
            pl.BlockSpec((F,), lambda i: (0,)),
        ],
        out_specs=pl.BlockSpec((N // 10, F), lambda i: (i, 0)),
        out_shape=jax.ShapeDtypeStruct((N, F), jnp.float32),
    )(partials, b)

    return (out, (edge_index, edge_weight))
